# baseline (speedup 1.0000x reference)
import jax
import jax.numpy as jnp
from jax import lax
from jax.experimental import pallas as pl
from jax.experimental.pallas import tpu as pltpu

_BIG = 1e30


def _pair_body(sz1_ref, sz2_ref, v1_ref, v2_ref, out_ref):
    i = pl.program_id(0)
    j = pl.program_id(1)
    x = v1_ref[0]
    y = v2_ref[0]
    L1 = x.shape[0]
    L2 = y.shape[0]
    x2 = jnp.sum(x * x, axis=1, keepdims=True)
    y2 = jnp.sum(y * y, axis=1, keepdims=True)
    yy = jnp.concatenate([y, y2], axis=1)
    xx = jnp.concatenate([-2.0 * x, jnp.ones((L1, 1), jnp.float32)], axis=1)
    g = lax.dot_general(xx, yy, (((1,), (1,)), ((), ())),
                        preferred_element_type=jnp.float32)
    d2 = jnp.maximum(x2 + g, 0.0)

    n1 = sz1_ref[i]
    n2 = sz2_ref[j]
    rows = lax.broadcasted_iota(jnp.int32, (L1, L2), 0)
    cols = lax.broadcasted_iota(jnp.int32, (L1, L2), 1)
    dm = jnp.where((rows < n1) & (cols < n2), d2, _BIG)
    minq = jnp.min(dm, axis=1, keepdims=True)
    minp = jnp.min(dm, axis=0, keepdims=True)
    rmask = lax.broadcasted_iota(jnp.int32, (L1, 1), 0) < n1
    cmask = lax.broadcasted_iota(jnp.int32, (1, L2), 1) < n2
    h1 = jnp.max(jnp.where(rmask, minq, -_BIG))
    h2 = jnp.max(jnp.where(cmask, minp, -_BIG))
    h = jnp.sqrt(jnp.maximum(h1, h2))
    out_ref[0, 0] = jnp.full((8, 128), h, jnp.float32)


def kernel(v1, sz1, v2, sz2):
    B1, L1, _ = v1.shape
    B2, L2, _ = v2.shape
    out = pl.pallas_call(
        _pair_body,
        grid=(B1, B2),
        in_specs=[
            pl.BlockSpec(memory_space=pltpu.SMEM),
            pl.BlockSpec(memory_space=pltpu.SMEM),
            pl.BlockSpec((1, L1, 3), lambda i, j: (i, 0, 0)),
            pl.BlockSpec((1, L2, 3), lambda i, j: (j, 0, 0)),
        ],
        out_specs=pl.BlockSpec((1, 1, 8, 128), lambda i, j: (i, j, 0, 0)),
        out_shape=jax.ShapeDtypeStruct((B1, B2, 8, 128), jnp.float32),
    )(sz1.astype(jnp.int32), sz2.astype(jnp.int32), v1, v2)
    return out[:, :, 0, 0]


# speedup vs baseline: 1.0021x; 1.0021x over previous
import jax
import jax.numpy as jnp
from jax import lax
from jax.experimental import pallas as pl
from jax.experimental.pallas import tpu as pltpu

_BIG = 1e30
_PAD = 1e17


def _pair_body(sz1_ref, sz2_ref, v1_ref, v2_ref, out_ref):
    i = pl.program_id(0)
    j = pl.program_id(1)
    x = v1_ref[0]
    y = v2_ref[0]
    L1 = x.shape[0]
    L2 = y.shape[0]
    x2 = jnp.sum(x * x, axis=1, keepdims=True)
    y2 = jnp.sum(y * y, axis=1, keepdims=True)
    yy = jnp.concatenate([y, y2], axis=1)
    xx = jnp.concatenate([-2.0 * x, jnp.ones((L1, 1), jnp.float32)], axis=1)
    g = lax.dot_general(xx, yy, (((1,), (1,)), ((), ())),
                        preferred_element_type=jnp.float32)
    d2 = x2 + g

    n1 = sz1_ref[i]
    n2 = sz2_ref[j]
    minq = jnp.min(d2, axis=1, keepdims=True)
    minp = jnp.min(d2, axis=0, keepdims=True)
    rmask = lax.broadcasted_iota(jnp.int32, (L1, 1), 0) < n1
    cmask = lax.broadcasted_iota(jnp.int32, (1, L2), 1) < n2
    h1 = jnp.max(jnp.where(rmask, minq, -_BIG))
    h2 = jnp.max(jnp.where(cmask, minp, -_BIG))
    h = jnp.sqrt(jnp.maximum(jnp.maximum(h1, h2), 0.0))
    out_ref[0, 0] = jnp.full((8, 128), h, jnp.float32)


def kernel(v1, sz1, v2, sz2):
    B1, L1, _ = v1.shape
    B2, L2, _ = v2.shape
    m1 = jnp.arange(L1)[None, :, None] < sz1[:, None, None]
    m2 = jnp.arange(L2)[None, :, None] < sz2[:, None, None]
    v1p = jnp.where(m1, v1, _PAD)
    v2p = jnp.where(m2, v2, _PAD)
    out = pl.pallas_call(
        _pair_body,
        grid=(B1, B2),
        in_specs=[
            pl.BlockSpec(memory_space=pltpu.SMEM),
            pl.BlockSpec(memory_space=pltpu.SMEM),
            pl.BlockSpec((1, L1, 3), lambda i, j: (i, 0, 0)),
            pl.BlockSpec((1, L2, 3), lambda i, j: (j, 0, 0)),
        ],
        out_specs=pl.BlockSpec((1, 1, 8, 128), lambda i, j: (i, j, 0, 0)),
        out_shape=jax.ShapeDtypeStruct((B1, B2, 8, 128), jnp.float32),
    )(sz1.astype(jnp.int32), sz2.astype(jnp.int32), v1p, v2p)
    return out[:, :, 0, 0]


# speedup vs baseline: 2.3446x; 2.3396x over previous
import jax
import jax.numpy as jnp
from jax import lax
from jax.experimental import pallas as pl
from jax.experimental.pallas import tpu as pltpu

_BIG = 1e30
_PAD = 1e17


def _row_body(sz1_ref, sz2_ref, v1_ref, v2_ref, out_ref):
    i = pl.program_id(0)
    x = v1_ref[0]
    yf = v2_ref[...]
    L1 = x.shape[0]
    B2 = out_ref.shape[1]
    L2 = yf.shape[0] // B2
    x2 = jnp.sum(x * x, axis=1, keepdims=True)
    y2 = jnp.sum(yf * yf, axis=1, keepdims=True)
    yy = jnp.concatenate([yf, y2], axis=1)
    xx = jnp.concatenate([-2.0 * x, jnp.ones((L1, 1), jnp.float32)], axis=1)
    g = lax.dot_general(xx, yy, (((1,), (1,)), ((), ())),
                        preferred_element_type=jnp.float32)
    d2 = x2 + g

    n1 = sz1_ref[i]
    rmask = lax.broadcasted_iota(jnp.int32, (L1, 1), 0) < n1
    cios = lax.broadcasted_iota(jnp.int32, (1, L2), 1)
    for j in range(B2):
        dj = d2[:, j * L2:(j + 1) * L2]
        n2 = sz2_ref[j]
        minq = jnp.min(dj, axis=1, keepdims=True)
        minp = jnp.min(dj, axis=0, keepdims=True)
        h1 = jnp.max(jnp.where(rmask, minq, -_BIG))
        h2 = jnp.max(jnp.where(cios < n2, minp, -_BIG))
        h = jnp.sqrt(jnp.maximum(jnp.maximum(h1, h2), 0.0))
        out_ref[0, j] = jnp.full((8, 128), h, jnp.float32)


def kernel(v1, sz1, v2, sz2):
    B1, L1, _ = v1.shape
    B2, L2, _ = v2.shape
    m1 = jnp.arange(L1)[None, :, None] < sz1[:, None, None]
    m2 = jnp.arange(L2)[None, :, None] < sz2[:, None, None]
    v1p = jnp.where(m1, v1, _PAD)
    v2p = jnp.where(m2, v2, _PAD).reshape(B2 * L2, 3)
    out = pl.pallas_call(
        _row_body,
        grid=(B1,),
        in_specs=[
            pl.BlockSpec(memory_space=pltpu.SMEM),
            pl.BlockSpec(memory_space=pltpu.SMEM),
            pl.BlockSpec((1, L1, 3), lambda i: (i, 0, 0)),
            pl.BlockSpec((B2 * L2, 3), lambda i: (0, 0)),
        ],
        out_specs=pl.BlockSpec((1, B2, 8, 128), lambda i: (i, 0, 0, 0)),
        out_shape=jax.ShapeDtypeStruct((B1, B2, 8, 128), jnp.float32),
    )(sz1.astype(jnp.int32), sz2.astype(jnp.int32), v1p, v2p)
    return out[:, :, 0, 0]
